# baseline (speedup 1.0000x reference)
import functools
import math
import jax
import jax.numpy as jnp
from jax.experimental import pallas as pl
from jax.experimental.pallas import tpu as pltpu


def _round_up(x, m):
    return (x + m - 1) // m * m


def _gcn_kernel(*refs, kchunk, n_streams, n_chunks):
    h_ref, w_ref, normk_ref = refs[:3]
    a_refs = refs[3 : 3 + n_streams]
    norm_ref, b_ref, o_ref, hw_ref, acc_ref = refs[3 + n_streams :]
    i = pl.program_id(0)
    c = pl.program_id(1)

    @pl.when((i == 0) & (c == 0))
    def _():
        hw = jnp.dot(h_ref[...], w_ref[...], preferred_element_type=jnp.float32)
        hw_ref[...] = (hw * normk_ref[...]).astype(hw_ref.dtype)

    part = jnp.dot(
        a_refs[0][...].astype(jnp.bfloat16),
        hw_ref[pl.ds(c * (n_streams * kchunk), kchunk), :],
        preferred_element_type=jnp.float32,
    )
    for s in range(1, n_streams):
        part += jnp.dot(
            a_refs[s][...].astype(jnp.bfloat16),
            hw_ref[pl.ds(c * (n_streams * kchunk) + s * kchunk, kchunk), :],
            preferred_element_type=jnp.float32,
        )

    @pl.when(c == 0)
    def _():
        acc_ref[...] = part

    @pl.when(c != 0)
    def _():
        acc_ref[...] += part

    @pl.when(c == n_chunks - 1)
    def _():
        tile_m = a_refs[0].shape[0]
        o_ref[pl.ds(i * tile_m, tile_m), :] = jnp.maximum(
            acc_ref[...] * norm_ref[...] + b_ref[...], 0.0
        ).astype(o_ref.dtype)


def kernel(h, weight, norm, adj, bias, *, tile_m=512, n_streams=1, n_chunks=2):
    N, in_feats = h.shape
    out_feats = weight.shape[1]

    norm = norm.reshape(N, 1).astype(jnp.float32)

    f_pad = _round_up(max(out_feats, 128), 128)
    n_pad = _round_up(N, math.lcm(tile_m, n_chunks * n_streams * 128))

    w_pad = jnp.zeros((in_feats, f_pad), jnp.float32).at[:, :out_feats].set(weight)
    b_pad = jnp.zeros((1, f_pad), jnp.float32).at[:, :out_feats].set(
        bias.reshape(1, -1)
    )
    if n_pad != N:
        h_p = jnp.zeros((n_pad, in_feats), jnp.float32).at[:N, :].set(h)
        norm_p = jnp.zeros((n_pad, 1), jnp.float32).at[:N, :].set(norm)
        a_p = jnp.zeros((n_pad, n_pad), jnp.float32).at[:N, :N].set(adj)
    else:
        h_p, norm_p, a_p = h.astype(jnp.float32), norm, adj.astype(jnp.float32)

    kchunk = n_pad // (n_chunks * n_streams)
    a_specs = [
        pl.BlockSpec(
            (tile_m, kchunk),
            functools.partial(lambda s, i, c: (i, c * n_streams + s), s),
        )
        for s in range(n_streams)
    ]
    out_p = pl.pallas_call(
        functools.partial(
            _gcn_kernel, kchunk=kchunk, n_streams=n_streams, n_chunks=n_chunks
        ),
        out_shape=jax.ShapeDtypeStruct((n_pad, f_pad), jnp.float32),
        grid=(n_pad // tile_m, n_chunks),
        in_specs=[
            pl.BlockSpec((n_pad, in_feats), lambda i, c: (0, 0)),
            pl.BlockSpec((in_feats, f_pad), lambda i, c: (0, 0)),
            pl.BlockSpec((n_pad, 1), lambda i, c: (0, 0)),
        ]
        + a_specs
        + [
            pl.BlockSpec((tile_m, 1), lambda i, c: (i, 0)),
            pl.BlockSpec((1, f_pad), lambda i, c: (0, 0)),
        ],
        out_specs=pl.BlockSpec((n_pad, f_pad), lambda i, c: (0, 0)),
        scratch_shapes=[
            pltpu.VMEM((n_pad, f_pad), jnp.bfloat16),
            pltpu.VMEM((tile_m, f_pad), jnp.float32),
        ],
        compiler_params=pltpu.CompilerParams(
            dimension_semantics=("arbitrary", "arbitrary")
        ),
    )(h_p, w_pad, norm_p, *([a_p] * n_streams), norm_p, b_pad)

    return out_p[:N, :out_feats]


# speedup vs baseline: 1.0165x; 1.0165x over previous
import functools
import math
import jax
import jax.numpy as jnp
from jax.experimental import pallas as pl
from jax.experimental.pallas import tpu as pltpu


def _round_up(x, m):
    return (x + m - 1) // m * m


def _gcn_kernel(*refs, kchunk, n_streams, n_chunks):
    h_ref, w_ref, normk_ref = refs[:3]
    a_refs = refs[3 : 3 + n_streams]
    norm_ref, b_ref, o_ref, hw_ref, acc_ref = refs[3 + n_streams :]
    i = pl.program_id(0)
    c = pl.program_id(1)

    @pl.when((i == 0) & (c == 0))
    def _():
        hw = jnp.dot(h_ref[...], w_ref[...], preferred_element_type=jnp.float32)
        hw_ref[...] = (hw * normk_ref[...]).astype(hw_ref.dtype)

    part = jnp.dot(
        a_refs[0][...].astype(jnp.bfloat16),
        hw_ref[pl.ds(c * (n_streams * kchunk), kchunk), :],
        preferred_element_type=jnp.float32,
    )
    for s in range(1, n_streams):
        part += jnp.dot(
            a_refs[s][...].astype(jnp.bfloat16),
            hw_ref[pl.ds(c * (n_streams * kchunk) + s * kchunk, kchunk), :],
            preferred_element_type=jnp.float32,
        )

    @pl.when(c == 0)
    def _():
        acc_ref[...] = part

    @pl.when(c != 0)
    def _():
        acc_ref[...] += part

    @pl.when(c == n_chunks - 1)
    def _():
        tile_m = a_refs[0].shape[0]
        o_ref[pl.ds(i * tile_m, tile_m), :] = jnp.maximum(
            acc_ref[...] * norm_ref[...] + b_ref[...], 0.0
        ).astype(o_ref.dtype)


def kernel(h, weight, norm, adj, bias, *, tile_m=512, n_streams=2, n_chunks=2):
    N, in_feats = h.shape
    out_feats = weight.shape[1]

    norm = norm.reshape(N, 1).astype(jnp.float32)

    f_pad = _round_up(max(out_feats, 128), 128)
    n_pad = _round_up(N, math.lcm(tile_m, n_chunks * n_streams * 128))

    w_pad = jnp.zeros((in_feats, f_pad), jnp.float32).at[:, :out_feats].set(weight)
    b_pad = jnp.zeros((1, f_pad), jnp.float32).at[:, :out_feats].set(
        bias.reshape(1, -1)
    )
    if n_pad != N:
        h_p = jnp.zeros((n_pad, in_feats), jnp.float32).at[:N, :].set(h)
        norm_p = jnp.zeros((n_pad, 1), jnp.float32).at[:N, :].set(norm)
        a_p = jnp.zeros((n_pad, n_pad), jnp.float32).at[:N, :N].set(adj)
    else:
        h_p, norm_p, a_p = h.astype(jnp.float32), norm, adj.astype(jnp.float32)

    kchunk = n_pad // (n_chunks * n_streams)
    a_specs = [
        pl.BlockSpec(
            (tile_m, kchunk),
            functools.partial(lambda s, i, c: (i, c * n_streams + s), s),
        )
        for s in range(n_streams)
    ]
    out_p = pl.pallas_call(
        functools.partial(
            _gcn_kernel, kchunk=kchunk, n_streams=n_streams, n_chunks=n_chunks
        ),
        out_shape=jax.ShapeDtypeStruct((n_pad, f_pad), jnp.float32),
        grid=(n_pad // tile_m, n_chunks),
        in_specs=[
            pl.BlockSpec((n_pad, in_feats), lambda i, c: (0, 0)),
            pl.BlockSpec((in_feats, f_pad), lambda i, c: (0, 0)),
            pl.BlockSpec((n_pad, 1), lambda i, c: (0, 0)),
        ]
        + a_specs
        + [
            pl.BlockSpec((tile_m, 1), lambda i, c: (i, 0)),
            pl.BlockSpec((1, f_pad), lambda i, c: (0, 0)),
        ],
        out_specs=pl.BlockSpec((n_pad, f_pad), lambda i, c: (0, 0)),
        scratch_shapes=[
            pltpu.VMEM((n_pad, f_pad), jnp.bfloat16),
            pltpu.VMEM((tile_m, f_pad), jnp.float32),
        ],
        compiler_params=pltpu.CompilerParams(
            dimension_semantics=("arbitrary", "arbitrary")
        ),
    )(h_p, w_pad, norm_p, *([a_p] * n_streams), norm_p, b_pad)

    return out_p[:N, :out_feats]


# speedup vs baseline: 1.1394x; 1.1209x over previous
import functools
import math
import jax
import jax.numpy as jnp
from jax.experimental import pallas as pl
from jax.experimental.pallas import tpu as pltpu


def _round_up(x, m):
    return (x + m - 1) // m * m


def _gcn_kernel(*refs, kchunk, n_streams):
    h_ref, w_ref, normk_ref = refs[:3]
    a_refs = refs[3 : 3 + n_streams]
    norm_ref, b_ref, o_ref, hw_ref = refs[3 + n_streams :]

    @pl.when(pl.program_id(0) == 0)
    def _():
        hw = jnp.dot(
            h_ref[...].astype(jnp.bfloat16),
            w_ref[...].astype(jnp.bfloat16),
            preferred_element_type=jnp.float32,
        )
        hw_ref[...] = (hw * normk_ref[...]).astype(hw_ref.dtype)

    acc = jnp.dot(
        a_refs[0][...].astype(jnp.bfloat16),
        hw_ref[:kchunk, :],
        preferred_element_type=jnp.float32,
    )
    for c in range(1, n_streams):
        acc += jnp.dot(
            a_refs[c][...].astype(jnp.bfloat16),
            hw_ref[c * kchunk : (c + 1) * kchunk, :],
            preferred_element_type=jnp.float32,
        )
    i = pl.program_id(0)
    tile_m = a_refs[0].shape[0]
    o_ref[pl.ds(i * tile_m, tile_m), :] = jnp.maximum(
        acc * norm_ref[...] + b_ref[...], 0.0
    ).astype(o_ref.dtype)


def kernel(h, weight, norm, adj, bias, *, tile_m=512, n_streams=2):
    N, in_feats = h.shape
    out_feats = weight.shape[1]

    norm = norm.reshape(N, 1).astype(jnp.float32)

    f_pad = _round_up(max(out_feats, 128), 128)
    n_pad = _round_up(N, math.lcm(tile_m, n_streams * 128))

    w_pad = jnp.zeros((in_feats, f_pad), jnp.float32).at[:, :out_feats].set(weight)
    b_pad = jnp.zeros((1, f_pad), jnp.float32).at[:, :out_feats].set(
        bias.reshape(1, -1)
    )
    if n_pad != N:
        h_p = jnp.zeros((n_pad, in_feats), jnp.float32).at[:N, :].set(h)
        norm_p = jnp.zeros((n_pad, 1), jnp.float32).at[:N, :].set(norm)
        a_p = jnp.zeros((n_pad, n_pad), jnp.float32).at[:N, :N].set(adj)
    else:
        h_p, norm_p, a_p = h.astype(jnp.float32), norm, adj.astype(jnp.float32)

    kchunk = n_pad // n_streams
    a_specs = [
        pl.BlockSpec((tile_m, kchunk), functools.partial(lambda c, i: (i, c), c))
        for c in range(n_streams)
    ]
    out_p = pl.pallas_call(
        functools.partial(_gcn_kernel, kchunk=kchunk, n_streams=n_streams),
        out_shape=jax.ShapeDtypeStruct((n_pad, f_pad), jnp.float32),
        grid=(n_pad // tile_m,),
        in_specs=[
            pl.BlockSpec((n_pad, in_feats), lambda i: (0, 0)),
            pl.BlockSpec((in_feats, f_pad), lambda i: (0, 0)),
            pl.BlockSpec((n_pad, 1), lambda i: (0, 0)),
        ]
        + a_specs
        + [
            pl.BlockSpec((tile_m, 1), lambda i: (i, 0)),
            pl.BlockSpec((1, f_pad), lambda i: (0, 0)),
        ],
        out_specs=pl.BlockSpec((n_pad, f_pad), lambda i: (0, 0)),
        scratch_shapes=[pltpu.VMEM((n_pad, f_pad), jnp.bfloat16)],
        compiler_params=pltpu.CompilerParams(dimension_semantics=("arbitrary",)),
    )(h_p, w_pad, norm_p, *([a_p] * n_streams), norm_p, b_pad)

    return out_p[:N, :out_feats]
